# Initial kernel scaffold; baseline (speedup 1.0000x reference)
import functools

import jax
import jax.numpy as jnp
from jax import lax
from jax.experimental import pallas as pl
from jax.experimental.pallas import tpu as pltpu
from jax.experimental.pallas import tpu_sc as plsc

_NBUF = 2
_CHUNK = 640


@functools.cache
def _build(B, D, C):
    info = plsc.get_sparse_core_info()
    nw = info.num_cores * info.num_subcores
    b_per_w = B // nw
    k = C // 128
    n_chunks = b_per_w // C
    idx_rows_per_w = b_per_w // 128

    mesh = plsc.VectorSubcoreMesh(core_axis_name="c", subcore_axis_name="s")

    @functools.partial(
        pl.kernel,
        mesh=mesh,
        out_type=jax.ShapeDtypeStruct((B, D), jnp.float32),
        scratch_types=[
            pltpu.VMEM((_NBUF, k, 128), jnp.int32),
            pltpu.VMEM((_NBUF, C, D), jnp.float32),
            pltpu.SemaphoreType.DMA((_NBUF,)),
            pltpu.SemaphoreType.DMA((_NBUF,)),
        ],
    )
    def kern(tid_hbm, table_hbm, out_hbm, idx_v, rows_v, gsem, wsem):
        wid = lax.axis_index("s") * info.num_cores + lax.axis_index("c")
        idx_row0 = wid * idx_rows_per_w
        out_row0 = wid * b_per_w

        def do_chunk(g, b):
            pltpu.sync_copy(tid_hbm.at[pl.ds(idx_row0 + g * k, k)], idx_v.at[b])
            cps = [
                pltpu.async_copy(
                    table_hbm.at[idx_v.at[b, j]],
                    rows_v.at[b, pl.ds(j * 128, 128)],
                    gsem.at[b],
                )
                for j in range(k)
            ]
            for cp in cps:
                cp.wait()
            pltpu.async_copy(
                rows_v.at[b], out_hbm.at[pl.ds(out_row0 + g * C, C)], wsem.at[b]
            )

        @pl.loop(0, n_chunks, step=_NBUF)
        def _chunks(g0):
            for b in range(_NBUF):
                g = g0 + b

                @pl.when(g >= _NBUF)
                def _():
                    pltpu.make_async_copy(
                        rows_v.at[b], out_hbm.at[pl.ds(0, C)], wsem.at[b]
                    ).wait()

                do_chunk(g, b)

        for b in range(_NBUF):
            pltpu.make_async_copy(
                rows_v.at[b], out_hbm.at[pl.ds(0, C)], wsem.at[b]
            ).wait()

    return kern


def kernel(token_ids, weight):
    n, s = token_ids.shape
    B = n * s
    D = weight.shape[1]
    tid2d = token_ids.reshape(B // 128, 128)
    out = _build(B, D, _CHUNK)(tid2d, weight)
    return out.reshape(n, s, D)


# speedup vs baseline: 1.8584x; 1.8584x over previous
import functools

import jax
import jax.numpy as jnp
from jax import lax
from jax.experimental import pallas as pl
from jax.experimental.pallas import tpu as pltpu
from jax.experimental.pallas import tpu_sc as plsc

_NBUF = 2
_CHUNK = 640


@functools.cache
def _build(B, D, C):
    info = plsc.get_sparse_core_info()
    nw = info.num_cores * info.num_subcores
    b_per_w = B // nw
    k = C // 128
    n_chunks = b_per_w // C

    mesh = plsc.VectorSubcoreMesh(core_axis_name="c", subcore_axis_name="s")

    @functools.partial(
        pl.kernel,
        mesh=mesh,
        out_type=jax.ShapeDtypeStruct((B, D), jnp.float32),
        compiler_params=pltpu.CompilerParams(use_tc_tiling_on_sc=False),
        scratch_types=[
            pltpu.VMEM((_NBUF, C), jnp.int32),
            pltpu.VMEM((_NBUF, C, D), jnp.float32),
            pltpu.SemaphoreType.DMA((_NBUF,)),
            pltpu.SemaphoreType.DMA((_NBUF,)),
        ],
    )
    def kern(tid_hbm, table_hbm, out_hbm, idx_v, rows_v, gsem, wsem):
        wid = lax.axis_index("s") * info.num_cores + lax.axis_index("c")
        out_row0 = wid * b_per_w

        def do_chunk(g, b):
            pltpu.sync_copy(
                tid_hbm.at[pl.ds(out_row0 + g * C, C)], idx_v.at[b]
            )
            cps = [
                pltpu.async_copy(
                    table_hbm.at[idx_v.at[b, pl.ds(j * 128, 128)]],
                    rows_v.at[b, pl.ds(j * 128, 128)],
                    gsem.at[b],
                )
                for j in range(k)
            ]
            for cp in cps:
                cp.wait()
            pltpu.async_copy(
                rows_v.at[b], out_hbm.at[pl.ds(out_row0 + g * C, C)], wsem.at[b]
            )

        @pl.loop(0, n_chunks, step=_NBUF)
        def _chunks(g0):
            for b in range(_NBUF):
                g = g0 + b

                @pl.when(g >= _NBUF)
                def _():
                    pltpu.make_async_copy(
                        rows_v.at[b], out_hbm.at[pl.ds(0, C)], wsem.at[b]
                    ).wait()

                do_chunk(g, b)

        for b in range(_NBUF):
            pltpu.make_async_copy(
                rows_v.at[b], out_hbm.at[pl.ds(0, C)], wsem.at[b]
            ).wait()

    return kern


def kernel(token_ids, weight):
    n, s = token_ids.shape
    B = n * s
    D = weight.shape[1]
    tid_flat = token_ids.reshape(B)
    out = _build(B, D, _CHUNK)(tid_flat, weight)
    return out.reshape(n, s, D)


# speedup vs baseline: 1.8772x; 1.0101x over previous
import functools

import jax
import jax.numpy as jnp
from jax import lax
from jax.experimental import pallas as pl
from jax.experimental.pallas import tpu as pltpu
from jax.experimental.pallas import tpu_sc as plsc

_NBUF = 2
_CHUNK = 512
_IDXW = 512


@functools.cache
def _build(B, D, C):
    info = plsc.get_sparse_core_info()
    nw = info.num_cores * info.num_subcores
    b_per_w = B // nw
    k = C // _IDXW
    n_chunks = b_per_w // C
    assert n_chunks % _NBUF == 0

    mesh = plsc.VectorSubcoreMesh(core_axis_name="c", subcore_axis_name="s")

    @functools.partial(
        pl.kernel,
        mesh=mesh,
        out_type=jax.ShapeDtypeStruct((B, D), jnp.float32),
        compiler_params=pltpu.CompilerParams(use_tc_tiling_on_sc=False),
        scratch_types=[
            pltpu.VMEM((b_per_w,), jnp.int32),
            pltpu.VMEM((_NBUF, C, D), jnp.float32),
            pltpu.SemaphoreType.DMA((_NBUF,)),
            pltpu.SemaphoreType.DMA((_NBUF,)),
        ],
    )
    def kern(tid_hbm, table_hbm, out_hbm, idx_all, rows_v, gsem, wsem):
        wid = lax.axis_index("s") * info.num_cores + lax.axis_index("c")
        row0 = wid * b_per_w
        pltpu.sync_copy(tid_hbm.at[pl.ds(row0, b_per_w)], idx_all)

        def fire_gather(g, b):
            for j in range(k):
                pltpu.async_copy(
                    table_hbm.at[idx_all.at[pl.ds(g * C + j * _IDXW, _IDXW)]],
                    rows_v.at[b, pl.ds(j * _IDXW, _IDXW)],
                    gsem.at[b],
                )

        def drain_gather(b):
            pltpu.make_async_copy(
                table_hbm.at[pl.ds(0, C)], rows_v.at[b], gsem.at[b]
            ).wait()

        def wait_write(b):
            pltpu.make_async_copy(
                rows_v.at[b], out_hbm.at[pl.ds(0, C)], wsem.at[b]
            ).wait()

        for b in range(_NBUF):
            fire_gather(b, b)

        @pl.loop(0, n_chunks, step=_NBUF)
        def _chunks(g0):
            for b in range(_NBUF):
                g = g0 + b
                drain_gather(b)
                pltpu.async_copy(
                    rows_v.at[b], out_hbm.at[pl.ds(row0 + g * C, C)], wsem.at[b]
                )

                @pl.when(g + _NBUF < n_chunks)
                def _():
                    wait_write(b)
                    fire_gather(g + _NBUF, b)

        for b in range(_NBUF):
            wait_write(b)

    return kern


def kernel(token_ids, weight):
    n, s = token_ids.shape
    B = n * s
    D = weight.shape[1]
    tid_flat = token_ids.reshape(B)
    out = _build(B, D, _CHUNK)(tid_flat, weight)
    return out.reshape(n, s, D)
